# baseline (speedup 1.0000x reference)
import functools

import jax
import jax.numpy as jnp
from jax import lax
from jax.experimental import pallas as pl
from jax.experimental.pallas import tpu as pltpu
from jax.experimental.pallas import tpu_sc as plsc

PAD = 0
SMOOTHING = 0.1
CONFIDENCE = 1.0 - SMOOTHING

N_ROWS = 1024
N_COLS = 100000
LANES = 16
COLS_LG = N_COLS // LANES

NC, NS = 2, 16
NW = NC * NS
BPW = N_ROWS // NW
CH = BPW // LANES

def _sc_gather_body(x1_hbm, tgt_hbm, out_hbm, tgt_v, idx_v, val_v, sem):
    wid = lax.axis_index("s") * NC + lax.axis_index("c")
    base = wid * BPW
    pltpu.sync_copy(tgt_hbm.at[pl.ds(base, BPW)], tgt_v)
    for c in range(CH):
        t = tgt_v[pl.ds(c * LANES, LANES)]
        rows = base + c * LANES + lax.iota(jnp.int32, 16)
        idx_v[pl.ds(c * LANES, LANES)] = rows * N_COLS + t
    pltpu.async_copy(x1_hbm.at[idx_v], val_v, sem).wait()
    pltpu.sync_copy(val_v, out_hbm.at[pl.ds(base, BPW)])


@functools.cache
def _sc_gather():
    mesh = plsc.VectorSubcoreMesh(
        core_axis_name="c", subcore_axis_name="s", num_cores=NC, num_subcores=NS
    )
    return pl.kernel(
        _sc_gather_body,
        out_type=jax.ShapeDtypeStruct((N_ROWS,), jnp.float32),
        mesh=mesh,
        scratch_types=[
            pltpu.VMEM((BPW,), jnp.int32),
            pltpu.VMEM((BPW,), jnp.int32),
            pltpu.VMEM((BPW,), jnp.float32),
            pltpu.SemaphoreType.DMA,
        ],
    )


BR = 32
EPS = SMOOTHING / (N_COLS - 2)


def _tc_body(x_ref, t_ref, xt_ref, out_ref, acc_ref):
    r = pl.program_id(0)
    blk = x_ref[...]
    s = jnp.sum(blk, axis=1, keepdims=True)
    x0 = blk[:, 0:1]
    t = t_ref[...]
    xt = xt_ref[...]
    row = EPS * (s - x0 - xt) + CONFIDENCE * xt
    row = jnp.where(t != PAD, row, 0.0)
    part = jnp.sum(row)

    @pl.when(r == 0)
    def _():
        acc_ref[0] = 0.0

    acc_ref[0] += part

    @pl.when(r == pl.num_programs(0) - 1)
    def _():
        out_ref[0, 0] = -acc_ref[0] / N_ROWS


def kernel(x, target):
    target = target.astype(jnp.int32)
    x1 = x.reshape(N_ROWS * N_COLS)
    xt = _sc_gather()(x1, target)
    loss = pl.pallas_call(
        _tc_body,
        grid=(N_ROWS // BR,),
        in_specs=[
            pl.BlockSpec((BR, N_COLS), lambda r: (r, 0)),
            pl.BlockSpec((BR, 1), lambda r: (r, 0)),
            pl.BlockSpec((BR, 1), lambda r: (r, 0)),
        ],
        out_specs=pl.BlockSpec(memory_space=pltpu.SMEM),
        out_shape=jax.ShapeDtypeStruct((1, 1), jnp.float32),
        scratch_shapes=[pltpu.SMEM((1,), jnp.float32)],
    )(x, target.reshape(N_ROWS, 1), xt.reshape(N_ROWS, 1))
    return loss[0, 0]


# speedup vs baseline: 2.2446x; 2.2446x over previous
import functools

import jax
import jax.numpy as jnp
from jax import lax
from jax.experimental import pallas as pl
from jax.experimental.pallas import tpu as pltpu
from jax.experimental.pallas import tpu_sc as plsc

PAD = 0
SMOOTHING = 0.1
CONFIDENCE = 1.0 - SMOOTHING

N_ROWS = 1024
N_COLS = 100000
LANES = 16
COLS_LG = N_COLS // LANES

NC, NS = 2, 16
NW = NC * NS
BPW = N_ROWS // NW
CH = BPW // LANES

def _sc_gather_body(x1_hbm, tgt_hbm, out_hbm, tgt_v, idx_v, val_v, sem):
    wid = lax.axis_index("s") * NC + lax.axis_index("c")
    base = wid * BPW
    pltpu.sync_copy(tgt_hbm.at[pl.ds(base, BPW)], tgt_v)
    for c in range(CH):
        t = tgt_v[pl.ds(c * LANES, LANES)]
        rows = base + c * LANES + lax.iota(jnp.int32, 16)
        idx_v[pl.ds(c * LANES, LANES)] = rows * N_COLS + t
    pltpu.async_copy(x1_hbm.at[idx_v], val_v, sem).wait()
    pltpu.sync_copy(val_v, out_hbm.at[pl.ds(base, BPW)])


@functools.cache
def _sc_gather():
    mesh = plsc.VectorSubcoreMesh(
        core_axis_name="c", subcore_axis_name="s", num_cores=NC, num_subcores=NS
    )
    return pl.kernel(
        _sc_gather_body,
        out_type=jax.ShapeDtypeStruct((N_ROWS,), jnp.float32),
        mesh=mesh,
        scratch_types=[
            pltpu.VMEM((BPW,), jnp.int32),
            pltpu.VMEM((BPW,), jnp.int32),
            pltpu.VMEM((BPW,), jnp.float32),
            pltpu.SemaphoreType.DMA,
        ],
    )


BR = 32
EPS = SMOOTHING / (N_COLS - 2)


def _tc_body(x_ref, t_ref, out_ref, acc_ref):
    r = pl.program_id(0)
    blk = x_ref[...]
    t = t_ref[...]
    cols = lax.broadcasted_iota(jnp.int32, (BR, N_COLS), 1)
    w = jnp.where(cols == t, jnp.float32(CONFIDENCE), jnp.float32(EPS))
    s = jnp.sum(blk * w, axis=1, keepdims=True)
    x0 = blk[:, 0:1]
    row = s - EPS * x0
    row = jnp.where(t != PAD, row, 0.0)
    part = jnp.sum(row)

    @pl.when(r == 0)
    def _():
        acc_ref[0] = 0.0

    acc_ref[0] += part

    @pl.when(r == pl.num_programs(0) - 1)
    def _():
        out_ref[0, 0] = -acc_ref[0] / N_ROWS


def kernel(x, target):
    target = target.astype(jnp.int32)
    loss = pl.pallas_call(
        _tc_body,
        grid=(N_ROWS // BR,),
        in_specs=[
            pl.BlockSpec((BR, N_COLS), lambda r: (r, 0)),
            pl.BlockSpec((BR, 1), lambda r: (r, 0)),
        ],
        out_specs=pl.BlockSpec(memory_space=pltpu.SMEM),
        out_shape=jax.ShapeDtypeStruct((1, 1), jnp.float32),
        scratch_shapes=[pltpu.SMEM((1,), jnp.float32)],
    )(x, target.reshape(N_ROWS, 1))
    return loss[0, 0]
